# Initial kernel scaffold; baseline (speedup 1.0000x reference)
import jax
import jax.numpy as jnp
from jax.experimental import pallas as pl


def kernel(pts, grid_0, grid_1, grid_2, grid_3):
    raise NotImplementedError("write your pallas kernel here")


# speedup vs baseline: 8.3099x; 8.3099x over previous
import functools

import jax
import jax.numpy as jnp
from jax import lax
from jax.experimental import pallas as pl
from jax.experimental.pallas import tpu as pltpu
from jax.experimental.pallas import tpu_sc as plsc

_NC, _NS, _L = 2, 16, 16
_NW = _NC * _NS
_B = 128
_FEAT = 8
_LODBITS = (4, 5, 6, 7)
_CORNERS = [(dx, dy, dz) for dx in (0, 1) for dy in (0, 1) for dz in (0, 1)]


@functools.lru_cache(maxsize=None)
def _make_kernel(n_pad: int):
    chunk = n_pad // _NW
    nblk = chunk // _B
    mesh = plsc.VectorSubcoreMesh(core_axis_name="c", subcore_axis_name="s")

    def body(xh, yh, zh, g0, g1, g2, g3, out, xv, yv, zv, fxf, fyf, fzf,
             idxf, rows2, outv, s0, s1, s2, s3):
        grids = [g0, g1, g2, g3]
        sems = [s0, s1, s2, s3]
        wid = lax.axis_index("s") * _NC + lax.axis_index("c")
        base = wid * chunk
        for h, v in ((xh, xv), (yh, yv), (zh, zv)):
            pltpu.sync_copy(h.at[pl.ds(base, chunk)], v)

        iota = lax.iota(jnp.int32, _L)
        rep8 = iota >> 3
        feat8 = iota & 7

        def block(j, carry):
            p0 = j * _B
            handles = []
            for l in range(4):
                lb = _LODBITS[l]
                lod = 1 << lb
                scale = (lod - 1) * 0.5
                for i in range(_B // _L):
                    sl = pl.ds(p0 + i * _L, _L)
                    vsl = pl.ds(i * _L, _L)
                    x = xv[sl] * scale + scale
                    y = yv[sl] * scale + scale
                    z = zv[sl] * scale + scale
                    xi = jnp.minimum(x.astype(jnp.int32), lod - 2)
                    yi = jnp.minimum(y.astype(jnp.int32), lod - 2)
                    zi = jnp.minimum(z.astype(jnp.int32), lod - 2)
                    fsl = pl.ds(l * _B + i * _L, _L)
                    fxf[fsl] = x - xi.astype(jnp.float32)
                    fyf[fsl] = y - yi.astype(jnp.float32)
                    fzf[fsl] = z - zi.astype(jnp.float32)
                    bidx = (xi << (2 * lb)) + (yi << lb) + zi
                    for c, (dx, dy, dz) in enumerate(_CORNERS):
                        off = dx * lod * lod + dy * lod + dz
                        idxf[pl.ds((l * 8 + c) * _B + i * _L, _L)] = bidx + off
                hs = []
                for c in range(8):
                    hs.append(pltpu.async_copy(
                        grids[l].at[idxf.at[pl.ds((l * 8 + c) * _B, _B)]],
                        rows2.at[pl.ds((l * 8 + c) * _B, _B)], sems[l]))
                handles.append(hs)

            for l in range(4):
                for h in handles[l]:
                    h.wait()
                opatt = feat8 + (rep8 << 5) + l * 8

                def acc_body(q, c2, l=l, opatt=opatt):
                    pvec = q * 2 + rep8
                    fvec = l * _B + pvec
                    fxe = plsc.load_gather(fxf, [fvec])
                    fye = plsc.load_gather(fyf, [fvec])
                    fze = plsc.load_gather(fzf, [fvec])
                    gxe = 1.0 - fxe
                    gye = 1.0 - fye
                    gze = 1.0 - fze
                    wxy = ((gxe * gye, gxe * fye), (fxe * gye, fxe * fye))
                    acc = None
                    for c, (dx, dy, dz) in enumerate(_CORNERS):
                        rv = plsc.load_gather(
                            rows2, [(l * 8 + c) * _B + pvec, feat8])
                        wc = wxy[dx][dy] * (fze if dz else gze)
                        acc = wc * rv if acc is None else acc + wc * rv
                    plsc.store_scatter(outv, [q * 64 + opatt], acc)
                    return c2

                lax.fori_loop(0, _B // 2, acc_body, 0)

            pltpu.sync_copy(outv, out.at[pl.ds((base + p0) * 32, _B * 32)])
            return carry

        lax.fori_loop(0, nblk, block, 0)

    return pl.kernel(
        body,
        out_type=jax.ShapeDtypeStruct((n_pad * 32,), jnp.float32),
        mesh=mesh,
        compiler_params=pltpu.CompilerParams(needs_layout_passes=False, use_tc_tiling_on_sc=False),
        scratch_types=[
            pltpu.VMEM((chunk,), jnp.float32),
            pltpu.VMEM((chunk,), jnp.float32),
            pltpu.VMEM((chunk,), jnp.float32),
            pltpu.VMEM((4 * _B,), jnp.float32),
            pltpu.VMEM((4 * _B,), jnp.float32),
            pltpu.VMEM((4 * _B,), jnp.float32),
            pltpu.VMEM((4 * 8 * _B,), jnp.int32),
            pltpu.VMEM((4 * 8 * _B, _FEAT), jnp.float32),
            pltpu.VMEM((_B * 32,), jnp.float32),
            pltpu.SemaphoreType.DMA,
            pltpu.SemaphoreType.DMA,
            pltpu.SemaphoreType.DMA,
            pltpu.SemaphoreType.DMA,
        ],
    )


def kernel(pts, grid_0, grid_1, grid_2, grid_3):
    n = pts.shape[0]
    tile = _B * _NW
    n_pad = -(-n // tile) * tile
    pad = (0, n_pad - n)
    xh = jnp.pad(pts[:, 0], pad)
    yh = jnp.pad(pts[:, 1], pad)
    zh = jnp.pad(pts[:, 2], pad)
    out = _make_kernel(n_pad)(xh, yh, zh, grid_0, grid_1, grid_2, grid_3)
    return out.reshape(n_pad, 32)[:n]
